# speedup vs baseline: 1.1201x; 1.1201x over previous
import functools

import jax
import jax.numpy as jnp
from jax.experimental import pallas as pl
from jax.experimental.pallas import tpu as pltpu

N_NODES = 10000
N_GRAPHS = 512
_MB = 400


def _proj_body(x_ref, w_ref, b_ref, o_ref):
    o_ref[...] = (
        jnp.dot(x_ref[...], w_ref[...], preferred_element_type=jnp.float32)
        + b_ref[...]
    )


def _proj(x, w, b):
    m, k = x.shape
    n = w.shape[1]
    grid = (m // _MB,)
    return pl.pallas_call(
        _proj_body,
        grid=grid,
        in_specs=[
            pl.BlockSpec((_MB, k), lambda i: (i, 0)),
            pl.BlockSpec((k, n), lambda i: (0, 0)),
            pl.BlockSpec((1, n), lambda i: (0, 0)),
        ],
        out_specs=pl.BlockSpec((_MB, n), lambda i: (i, 0)),
        out_shape=jax.ShapeDtypeStruct((m, n), jnp.float32),
    )(x, w, b.reshape(1, n))


def _tconv(x, src, dst, edge_attr, p, n_nodes):
    wqkvs = jnp.concatenate([p['Wq'], p['Wk'], p['Wv'], p['Wskip']], axis=1)
    bqkvs = jnp.concatenate([p['bq'], p['bk'], p['bv'], p['bskip']], axis=0)
    qkvs = _proj(x, wqkvs, bqkvs)
    q = qkvs[:, :128]
    k = qkvs[:, 128:256]
    v = qkvs[:, 256:384]
    skip = qkvs[:, 384:]
    e = edge_attr @ p['We']
    kj = k[src] + e
    vj = v[src] + e
    qi = q[dst]
    scale = 1.0 / jnp.sqrt(jnp.float32(q.shape[-1]))
    logits = jnp.sum(qi * kj, axis=-1) * scale
    m = jax.ops.segment_max(logits, dst, num_segments=n_nodes)
    m = jnp.where(jnp.isfinite(m), m, 0.0)
    ex = jnp.exp(logits - m[dst])
    den = jax.ops.segment_sum(ex, dst, num_segments=n_nodes)
    alpha = ex / (den[dst] + 1e-16)
    agg = jax.ops.segment_sum(alpha[:, None] * vj, dst, num_segments=n_nodes)
    return agg + skip


@jax.jit
def kernel(x, edge_index, edge_attr, batch, params):
    src = edge_index[0]
    dst = edge_index[1]
    n = x.shape[0]
    h = jax.nn.leaky_relu(_tconv(x, src, dst, edge_attr, params['conv1'], n), 0.01)
    atom_embs = jax.nn.leaky_relu(_tconv(h, src, dst, edge_attr, params['conv2'], n), 0.01)
    pooled = jax.ops.segment_sum(atom_embs, batch, num_segments=N_GRAPHS)
    norm = jnp.linalg.norm(pooled, axis=1, keepdims=True)
    pooled = pooled / jnp.maximum(norm, 1e-12)
    h2 = jax.nn.leaky_relu(pooled @ params['fc1']['W'] + params['fc1']['b'], 0.01)
    out = h2 @ params['fc2']['W'] + params['fc2']['b']
    return (out, atom_embs)


# speedup vs baseline: 1.8246x; 1.6290x over previous
import functools

import jax
import jax.numpy as jnp
from jax import lax
from jax.experimental import pallas as pl
from jax.experimental.pallas import tpu as pltpu
from jax.experimental.pallas import tpu_sc as plsc

N = 10000
NPAD = 10240
E = 320000
NW = 32
BR = NPAD // NW
EW = E // NW
CH = 80
NCH = EW // CH
EP = E + NW * NW * 8
EPD = EP + 128
CAPB = 24576
LCH = 128
MAXCH = CAPB // LCH
SCALE = 0.08838834764831845
SENT = 1 << 20
D = 128
DE = 16
NG = 512
MB = 400

MESH = plsc.VectorSubcoreMesh(core_axis_name="c", subcore_axis_name="s")
CPSC = pltpu.CompilerParams(needs_layout_passes=False)
CPSC2 = pltpu.CompilerParams(needs_layout_passes=False, use_tc_tiling_on_sc=False)


def _wid():
    return lax.axis_index("s") * 2 + lax.axis_index("c")


def _ceil8(x):
    return ((x + 7) >> 3) << 3


@functools.partial(
    pl.kernel, mesh=MESH, compiler_params=CPSC,
    out_type=jax.ShapeDtypeStruct((NW, NW), jnp.int32),
    scratch_types=[
        pltpu.VMEM((CH,), jnp.int32),
        pltpu.VMEM((NW,), jnp.int32),
    ],
)
def _sc_hist(dst_hbm, hist_hbm, dbuf, hloc):
    w = _wid()
    iota = lax.iota(jnp.int32, 16)
    z = jnp.zeros((16,), jnp.int32)
    hloc[pl.ds(0, 16)] = z
    hloc[pl.ds(16, 16)] = z
    ones = jnp.full((16,), 1, jnp.int32)

    def chunk(c, _):
        pltpu.sync_copy(dst_hbm.at[pl.ds(w * EW + c * CH, CH)], dbuf)
        for g in range(CH // 16):
            d16 = dbuf[pl.ds(g * 16, 16)]
            b16 = (d16 * 6554) >> 21
            plsc.addupdate_scatter(hloc, [b16], ones)
        return 0

    lax.fori_loop(0, NCH, chunk, 0)
    pltpu.sync_copy(hloc, hist_hbm.at[w])


@functools.partial(
    pl.kernel, mesh=MESH, compiler_params=CPSC,
    out_type=(jax.ShapeDtypeStruct((EPD,), jnp.int32),
              jax.ShapeDtypeStruct((EPD,), jnp.int32),
              jax.ShapeDtypeStruct((EPD,), jnp.int32)),
    scratch_types=[
        pltpu.VMEM((NW * NW,), jnp.int32),
        pltpu.VMEM((NW,), jnp.int32),
        pltpu.VMEM((NW,), jnp.int32),
        pltpu.VMEM((CH,), jnp.int32),
        pltpu.VMEM((CH,), jnp.int32),
        pltpu.VMEM((CH,), jnp.int32),
        pltpu.VMEM((CH,), jnp.int32),
        pltpu.VMEM((128,), jnp.int32),
        pltpu.VMEM((128,), jnp.int32),
        pltpu.VMEM((128,), jnp.int32),
        pltpu.SemaphoreType.DMA,
        pltpu.SemaphoreType.DMA,
        pltpu.SemaphoreType.DMA,
    ],
)
def _sc_bucket(src_hbm, dst_hbm, hist_hbm, bsrc_hbm, bdst_hbm, bperm_hbm,
               hflat, rc, sentp, sbuf, dbuf, pbuf, posb, spos, sdst, szero,
               sem1, sem2, sem3):
    w = _wid()
    iota = lax.iota(jnp.int32, 16)
    lane0 = iota == 0
    ones = jnp.full((16,), 1, jnp.int32)
    pltpu.sync_copy(hist_hbm, hflat)

    run = jnp.int32(0)
    for b in range(NW):
        h1 = plsc.load_gather(hflat, [iota * NW + b])
        h2 = plsc.load_gather(hflat, [(iota + 16) * NW + b])
        c1 = _ceil8(h1)
        c2 = _ceil8(h2)
        tot_b = jnp.sum(c1) + jnp.sum(c2)
        myoff = (jnp.sum(jnp.where(iota < w, c1, 0))
                 + jnp.sum(jnp.where(iota + 16 < w, c2, 0)))
        base_b = run + myoff
        hw = plsc.load_gather(hflat, [jnp.full((16,), w * NW + b, jnp.int32)])[0]
        capw = _ceil8(hw)
        plsc.store_scatter(rc, [jnp.full((16,), b, jnp.int32)],
                           jnp.full((16,), base_b, jnp.int32), mask=lane0)
        sp = jnp.where(capw > 0, base_b + capw - 8, jnp.int32(EP))
        plsc.store_scatter(sentp, [jnp.full((16,), b, jnp.int32)],
                           jnp.full((16,), sp, jnp.int32), mask=lane0)
        run = run + tot_b

    for ci in range(4):
        for j in range(8):
            b = ci * 8 + j
            sp = plsc.load_gather(sentp, [jnp.full((16,), b, jnp.int32)])[0]
            pos16 = jnp.where(iota < 8, sp + iota, jnp.int32(EP) + iota)
            spos[pl.ds(j * 16, 16)] = pos16
            sdst[pl.ds(j * 16, 16)] = jnp.full((16,), SENT, jnp.int32)
            szero[pl.ds(j * 16, 16)] = jnp.zeros((16,), jnp.int32)
        d1 = pltpu.async_copy(sdst, bdst_hbm.at[spos], sem1)
        d2 = pltpu.async_copy(szero, bsrc_hbm.at[spos], sem2)
        d3 = pltpu.async_copy(szero, bperm_hbm.at[spos], sem3)
        d1.wait(); d2.wait(); d3.wait()

    def chunk(c, _):
        ebase = w * EW + c * CH
        pltpu.sync_copy(src_hbm.at[pl.ds(ebase, CH)], sbuf)
        pltpu.sync_copy(dst_hbm.at[pl.ds(ebase, CH)], dbuf)
        for g in range(CH // 16):
            d16 = dbuf[pl.ds(g * 16, 16)]
            b16 = (d16 * 6554) >> 21
            cnt, _last = plsc.scan_count(b16)
            pos16 = plsc.load_gather(rc, [b16]) + (cnt - 1)
            plsc.addupdate_scatter(rc, [b16], ones)
            posb[pl.ds(g * 16, 16)] = pos16
            pbuf[pl.ds(g * 16, 16)] = ebase + g * 16 + iota
        d1 = pltpu.async_copy(sbuf, bsrc_hbm.at[posb], sem1)
        d2 = pltpu.async_copy(dbuf, bdst_hbm.at[posb], sem2)
        d3 = pltpu.async_copy(pbuf, bperm_hbm.at[posb], sem3)
        d1.wait(); d2.wait(); d3.wait()
        return 0

    lax.fori_loop(0, NCH, chunk, 0)


@functools.partial(
    pl.kernel, mesh=MESH, compiler_params=CPSC2,
    out_type=(jax.ShapeDtypeStruct((NPAD, D), jnp.float32),
              jax.ShapeDtypeStruct((NPAD, DE), jnp.float32)),
    scratch_types=[
        pltpu.VMEM((NW * NW,), jnp.int32),
        pltpu.VMEM((LCH,), jnp.int32),
        pltpu.VMEM((LCH,), jnp.int32),
        pltpu.VMEM((LCH,), jnp.int32),
        pltpu.VMEM((LCH,), jnp.int32),
        pltpu.VMEM((LCH, D), jnp.float32),
        pltpu.VMEM((LCH, D), jnp.float32),
        pltpu.VMEM((LCH, DE), jnp.float32),
        pltpu.VMEM((LCH, DE), jnp.float32),
        pltpu.VMEM((CAPB,), jnp.float32),
        pltpu.VMEM((BR,), jnp.float32),
        pltpu.VMEM((BR,), jnp.float32),
        pltpu.VMEM((LCH,), jnp.float32),
        pltpu.VMEM((BR, D), jnp.float32),
        pltpu.VMEM((BR, DE), jnp.float32),
        pltpu.SemaphoreType.DMA,
        pltpu.SemaphoreType.DMA,
        pltpu.SemaphoreType.DMA,
        pltpu.SemaphoreType.DMA,
    ],
)
def _sc_conv(q_hbm, k_hbm, v_hbm, qe_hbm, ea_hbm, bsrc_hbm, bdst_hbm,
             bperm_hbm, hist_hbm, aggv_hbm, aggea_hbm,
             hflat, draw, idxs, idxp, idxd, qrows, krows, qerows, earows,
             lg, mloc, denloc, abuf, aggv, aggea,
             sem1, sem2, sem3, sem4):
    w = _wid()
    iota = lax.iota(jnp.int32, 16)
    lo = w * BR
    hi = lo + BR
    pltpu.sync_copy(hist_hbm, hflat)

    rstart = jnp.int32(0)
    rlen = jnp.int32(0)
    for b in range(NW):
        c1 = _ceil8(plsc.load_gather(hflat, [iota * NW + b]))
        c2 = _ceil8(plsc.load_gather(hflat, [(iota + 16) * NW + b]))
        tot_b = jnp.sum(c1) + jnp.sum(c2)
        rstart = rstart + jnp.where(jnp.int32(b) < w, tot_b, 0)
        rlen = rlen + jnp.where(jnp.int32(b) == w, tot_b, 0)
    nch = jnp.minimum((rlen + (LCH - 1)) >> 7, MAXCH)

    zf = jnp.zeros((16,), jnp.float32)
    for t in range(BR // 16):
        mloc[pl.ds(t * 16, 16)] = zf - 1e30
        denloc[pl.ds(t * 16, 16)] = zf

    def zrow(r, _):
        for jb in range(D // 16):
            aggv[r, pl.ds(jb * 16, 16)] = zf
        aggea[r, :] = zf
        return 0
    lax.fori_loop(0, BR, zrow, 0)

    def load_meta(cb, need_perm):
        pltpu.sync_copy(bdst_hbm.at[pl.ds(cb, LCH)], draw)
        pltpu.sync_copy(bsrc_hbm.at[pl.ds(cb, LCH)], idxs)
        if need_perm:
            pltpu.sync_copy(bperm_hbm.at[pl.ds(cb, LCH)], idxp)

    def group_mask(c, g):
        d16 = draw[pl.ds(g * 16, 16)]
        eidx = c * LCH + g * 16 + iota
        valid = (d16 >= lo) & (d16 < hi) & (eidx < rlen)
        return d16, valid

    def passA(c, _):
        cb = pl.multiple_of(rstart + c * LCH, 8)
        load_meta(cb, True)
        for g in range(LCH // 16):
            d16, valid = group_mask(c, g)
            idxd[pl.ds(g * 16, 16)] = jnp.where(valid, d16, lo)
            s16 = idxs[pl.ds(g * 16, 16)]
            idxs[pl.ds(g * 16, 16)] = jnp.where(valid, s16, 0)
            p16 = idxp[pl.ds(g * 16, 16)]
            idxp[pl.ds(g * 16, 16)] = jnp.where(valid, p16, 0)
        d1 = pltpu.async_copy(q_hbm.at[idxd], qrows, sem1)
        d2 = pltpu.async_copy(k_hbm.at[idxs], krows, sem2)
        d3 = pltpu.async_copy(qe_hbm.at[idxd], qerows, sem3)
        d4 = pltpu.async_copy(ea_hbm.at[idxp], earows, sem4)
        d1.wait(); d2.wait(); d3.wait(); d4.wait()

        accs0 = tuple(zf for _ in range(LCH // 16))

        def dotj(j, accs):
            jv = jnp.full((16,), j, jnp.int32)
            return tuple(
                accs[g] + plsc.load_gather(qrows, [g * 16 + iota, jv])
                * plsc.load_gather(krows, [g * 16 + iota, jv])
                for g in range(LCH // 16))
        accs = lax.fori_loop(0, D, dotj, accs0)

        def dotj2(j, accs):
            jv = jnp.full((16,), j, jnp.int32)
            return tuple(
                accs[g] + plsc.load_gather(qerows, [g * 16 + iota, jv])
                * plsc.load_gather(earows, [g * 16 + iota, jv])
                for g in range(LCH // 16))
        accs = lax.fori_loop(0, DE, dotj2, accs)

        for g in range(LCH // 16):
            d16, valid = group_mask(c, g)
            ldcl = jnp.where(valid, d16 - lo, 0)
            logit = accs[g] * SCALE
            cidx = jnp.full((16,), c * LCH + g * 16, jnp.int32) + iota
            plsc.store_scatter(lg, [cidx], logit)
            mg = plsc.load_gather(mloc, [ldcl], mask=valid)
            plsc.store_scatter(mloc, [ldcl], jnp.maximum(mg, logit),
                               mask=valid)
        return 0

    lax.fori_loop(0, nch, passA, 0)

    def passB(c, _):
        cb = pl.multiple_of(rstart + c * LCH, 8)
        pltpu.sync_copy(bdst_hbm.at[pl.ds(cb, LCH)], draw)
        for g in range(LCH // 16):
            d16, valid = group_mask(c, g)
            ldcl = jnp.where(valid, d16 - lo, 0)
            cidx = jnp.full((16,), c * LCH + g * 16, jnp.int32) + iota
            l16 = plsc.load_gather(lg, [cidx])
            mg = plsc.load_gather(mloc, [ldcl], mask=valid)
            ex = jnp.exp(l16 - mg)
            plsc.store_scatter(lg, [cidx], ex, mask=valid)
            plsc.addupdate_scatter(denloc, [ldcl], ex, mask=valid)
        return 0

    lax.fori_loop(0, nch, passB, 0)

    def passC(c, _):
        cb = pl.multiple_of(rstart + c * LCH, 8)
        load_meta(cb, True)
        for g in range(LCH // 16):
            d16, valid = group_mask(c, g)
            idxd[pl.ds(g * 16, 16)] = jnp.where(valid, d16, lo)
            s16 = idxs[pl.ds(g * 16, 16)]
            idxs[pl.ds(g * 16, 16)] = jnp.where(valid, s16, 0)
            p16 = idxp[pl.ds(g * 16, 16)]
            idxp[pl.ds(g * 16, 16)] = jnp.where(valid, p16, 0)
        d1 = pltpu.async_copy(v_hbm.at[idxs], qrows, sem1)
        d2 = pltpu.async_copy(ea_hbm.at[idxp], earows, sem2)
        d1.wait(); d2.wait()
        for g in range(LCH // 16):
            d16, valid = group_mask(c, g)
            ldcl = jnp.where(valid, d16 - lo, 0)
            cidx = jnp.full((16,), c * LCH + g * 16, jnp.int32) + iota
            ex = plsc.load_gather(lg, [cidx])
            den = plsc.load_gather(denloc, [ldcl], mask=valid)
            alpha = ex / (den + 1e-16)
            abuf[pl.ds(g * 16, 16)] = jnp.where(valid, alpha, 0.0)

        def edge(e, _):
            ev = jnp.full((16,), e, jnp.int32)
            a16 = plsc.load_gather(abuf, [ev])

            @pl.when(a16[0] != 0.0)
            def _():
                ld = plsc.load_gather(idxd, [ev])[0] - lo
                for jb in range(D // 16):
                    aggv[ld, pl.ds(jb * 16, 16)] = (
                        aggv[ld, pl.ds(jb * 16, 16)]
                        + qrows[e, pl.ds(jb * 16, 16)] * a16)
                aggea[ld, :] = aggea[ld, :] + earows[e, :] * a16
            return 0

        lax.fori_loop(0, LCH, edge, 0)
        return 0

    lax.fori_loop(0, nch, passC, 0)

    pltpu.sync_copy(aggv, aggv_hbm.at[pl.ds(lo, BR)])
    pltpu.sync_copy(aggea, aggea_hbm.at[pl.ds(lo, BR)])


def _proj_body(x_ref, wq, bq, wk, bk, wv, bv, ws, bs, we,
               q_ref, k_ref, v_ref, s_ref, qe_ref):
    xb = x_ref[...]
    q = jnp.dot(xb, wq[...], preferred_element_type=jnp.float32) + bq[...]
    q_ref[...] = q
    k_ref[...] = jnp.dot(xb, wk[...], preferred_element_type=jnp.float32) + bk[...]
    v_ref[...] = jnp.dot(xb, wv[...], preferred_element_type=jnp.float32) + bv[...]
    s_ref[...] = jnp.dot(xb, ws[...], preferred_element_type=jnp.float32) + bs[...]
    qe_ref[...] = lax.dot_general(q, we[...], (((1,), (1,)), ((), ())),
                                  preferred_element_type=jnp.float32)


def _tc_proj(x, p):
    grid = (N // MB,)
    wspec = pl.BlockSpec((D, D), lambda i: (0, 0))
    bspec = pl.BlockSpec((1, D), lambda i: (0, 0))
    outs = [jax.ShapeDtypeStruct((N, D), jnp.float32)] * 4 + [
        jax.ShapeDtypeStruct((N, DE), jnp.float32)]
    return pl.pallas_call(
        _proj_body,
        grid=grid,
        in_specs=[
            pl.BlockSpec((MB, D), lambda i: (i, 0)),
            wspec, bspec, wspec, bspec, wspec, bspec, wspec, bspec,
            pl.BlockSpec((DE, D), lambda i: (0, 0)),
        ],
        out_specs=[
            pl.BlockSpec((MB, D), lambda i: (i, 0)),
            pl.BlockSpec((MB, D), lambda i: (i, 0)),
            pl.BlockSpec((MB, D), lambda i: (i, 0)),
            pl.BlockSpec((MB, D), lambda i: (i, 0)),
            pl.BlockSpec((MB, DE), lambda i: (i, 0)),
        ],
        out_shape=outs,
    )(x, p['Wq'], p['bq'].reshape(1, D), p['Wk'], p['bk'].reshape(1, D),
      p['Wv'], p['bv'].reshape(1, D), p['Wskip'], p['bskip'].reshape(1, D),
      p['We'])


def _combine_body(av_ref, ae_ref, we_ref, sk_ref, o_ref):
    h = (av_ref[...]
         + jnp.dot(ae_ref[...], we_ref[...], preferred_element_type=jnp.float32)
         + sk_ref[...])
    o_ref[...] = jnp.where(h > 0, h, 0.01 * h)


def _tc_combine(aggv, aggea, we, skip):
    grid = (N // MB,)
    return pl.pallas_call(
        _combine_body,
        grid=grid,
        in_specs=[
            pl.BlockSpec((MB, D), lambda i: (i, 0)),
            pl.BlockSpec((MB, DE), lambda i: (i, 0)),
            pl.BlockSpec((DE, D), lambda i: (0, 0)),
            pl.BlockSpec((MB, D), lambda i: (i, 0)),
        ],
        out_specs=pl.BlockSpec((MB, D), lambda i: (i, 0)),
        out_shape=jax.ShapeDtypeStruct((N, D), jnp.float32),
    )(aggv, aggea, we, skip)


def _pool_body(x_ref, b_ref, o_ref):
    i = pl.program_id(0)

    @pl.when(i == 0)
    def _():
        o_ref[...] = jnp.zeros_like(o_ref)

    g = lax.broadcasted_iota(jnp.int32, (NG, MB), 0)
    oh = (b_ref[0] == g).astype(jnp.float32)
    o_ref[...] += jnp.dot(oh, x_ref[...], preferred_element_type=jnp.float32)


def _tc_pool(x, batch2d):
    grid = (N // MB,)
    return pl.pallas_call(
        _pool_body,
        grid=grid,
        in_specs=[
            pl.BlockSpec((MB, D), lambda i: (i, 0)),
            pl.BlockSpec((1, 1, MB), lambda i: (i, 0, 0)),
        ],
        out_specs=pl.BlockSpec((NG, D), lambda i: (0, 0)),
        out_shape=jax.ShapeDtypeStruct((NG, D), jnp.float32),
    )(x, batch2d)


def _head_body(p_ref, w1, b1, w2, b2, o_ref):
    pooled = p_ref[...]
    nrm = jnp.sqrt(jnp.sum(pooled * pooled, axis=1, keepdims=True))
    pooled = pooled / jnp.maximum(nrm, 1e-12)
    h2 = jnp.dot(pooled, w1[...], preferred_element_type=jnp.float32) + b1[...]
    h2 = jnp.where(h2 > 0, h2, 0.01 * h2)
    o_ref[...] = jnp.dot(h2, w2[...], preferred_element_type=jnp.float32) + b2[...]


def _tc_head(pooled, fc1, fc2):
    return pl.pallas_call(
        _head_body,
        in_specs=[
            pl.BlockSpec((NG, D), lambda: (0, 0)),
            pl.BlockSpec((D, D), lambda: (0, 0)),
            pl.BlockSpec((1, D), lambda: (0, 0)),
            pl.BlockSpec((D, D), lambda: (0, 0)),
            pl.BlockSpec((1, D), lambda: (0, 0)),
        ],
        out_specs=pl.BlockSpec((NG, D), lambda: (0, 0)),
        out_shape=jax.ShapeDtypeStruct((NG, D), jnp.float32),
    )(pooled, fc1['W'], fc1['b'].reshape(1, D), fc2['W'], fc2['b'].reshape(1, D))


def _layer(x, p, ea, bsrc, bdst, bperm, hist):
    q, k, v, skip, qe = _tc_proj(x, p)
    aggv, aggea = _sc_conv(q, k, v, qe, ea, bsrc, bdst, bperm, hist)
    return _tc_combine(aggv[:N], aggea[:N], p['We'], skip)


@jax.jit
def kernel(x, edge_index, edge_attr, batch, params):
    src = edge_index[0]
    dst = edge_index[1]
    hist = _sc_hist(dst).reshape(NW * NW)
    bsrc, bdst, bperm = _sc_bucket(src, dst, hist)
    h = _layer(x, params['conv1'], edge_attr, bsrc, bdst, bperm, hist)
    atom_embs = _layer(h, params['conv2'], edge_attr, bsrc, bdst, bperm, hist)
    pooled = _tc_pool(atom_embs, batch.reshape(N // MB, 1, MB))
    out = _tc_head(pooled, params['fc1'], params['fc2'])
    return (out, atom_embs)
